# baseline (speedup 1.0000x reference)
import functools

import jax
import jax.numpy as jnp
from jax import lax
from jax.experimental import pallas as pl
from jax.experimental.pallas import tpu as pltpu
from jax.experimental.pallas import tpu_sc as plsc

B_POS = 16384
B_NEG = 81920
B_TOT = B_POS + B_NEG
R_TAB = 1999999
D = 64
L = 16
IDX_W = 128
PK = 128


NC = 2
NS = 16
NW = NC * NS

NBLK = 15625
BLK_T = (NBLK + NW - 1) // NW
LAST_START = R_TAB - IDX_W

ROWS_W = B_TOT // NW
CH = 256
N_CH = ROWS_W // CH
G_CH = CH // L


SPLIT = 1000448
TBLK = 512
GRID_B = SPLIT // TBLK
NSRC_B = R_TAB // TBLK
SAFE_B = 1952
TAILS = SPLIT + SAFE_B * TBLK - TBLK


def _tc_repack(Ut, Vt, Utail, Vtail):
  def body(ut_ref, ub_ref, utl_ref, vt_ref, vb_ref, vtl_ref, uo_ref, vo_ref):
    b = pl.program_id(0)
    use_tail = b >= SAFE_B
    ub = jnp.where(use_tail, utl_ref[...], ub_ref[...])
    vb = jnp.where(use_tail, vtl_ref[...], vb_ref[...])
    uo_ref[...] = jnp.concatenate([ut_ref[...].T, ub.T], axis=1)
    vo_ref[...] = jnp.concatenate([vt_ref[...].T, vb.T], axis=1)

  top_spec = pl.BlockSpec((D, TBLK), lambda b: (0, b))
  bot_spec = pl.BlockSpec(
      (D, TBLK), lambda b: (0, jnp.minimum(GRID_B + b, NSRC_B - 1)))
  tail_spec = pl.BlockSpec(
      (D, TBLK), lambda b: (0, jnp.clip(b - (SAFE_B - 1), 0, 1)))
  out_spec = pl.BlockSpec((TBLK, PK), lambda b: (b, 0))
  return pl.pallas_call(
      body,
      grid=(GRID_B,),
      in_specs=[top_spec, bot_spec, tail_spec,
                top_spec, bot_spec, tail_spec],
      out_specs=[out_spec, out_spec],
      out_shape=[jax.ShapeDtypeStruct((SPLIT, PK), jnp.float32)] * 2,
  )(Ut, Ut, Utail, Vt, Vt, Vtail)


def _sc_scores(u_idx, v_idx, Upk, Vpk):
  mesh = plsc.VectorSubcoreMesh(core_axis_name="c", subcore_axis_name="s")

  @functools.partial(
      pl.kernel,
      out_type=jax.ShapeDtypeStruct((B_TOT,), jnp.float32),
      mesh=mesh,
      scratch_types=[
          pltpu.VMEM((ROWS_W,), jnp.int32),
          pltpu.VMEM((ROWS_W,), jnp.int32),
          pltpu.VMEM((ROWS_W,), jnp.int32),
          pltpu.VMEM((ROWS_W,), jnp.int32),
          pltpu.VMEM((CH, PK), jnp.float32),
          pltpu.VMEM((CH, PK), jnp.float32),
          pltpu.VMEM((ROWS_W,), jnp.float32),
          pltpu.SemaphoreType.DMA,
      ],
  )
  def k(u_idx_hbm, v_idx_hbm, u_hbm, v_hbm, out_hbm,
        uix, vix, urix, vrix, urows, vrows, sc, sem):
    wid = lax.axis_index("s") * NC + lax.axis_index("c")
    base = wid * ROWS_W
    pltpu.sync_copy(u_idx_hbm.at[pl.ds(base, ROWS_W)], uix)
    pltpu.sync_copy(v_idx_hbm.at[pl.ds(base, ROWS_W)], vix)

    @plsc.parallel_loop(0, ROWS_W // L, unroll=4)
    def _(t):
      s = t * L
      uu = uix[pl.ds(s, L)]
      vv = vix[pl.ds(s, L)]
      urix[pl.ds(s, L)] = jnp.where(uu >= SPLIT, uu - SPLIT, uu)
      vrix[pl.ds(s, L)] = jnp.where(vv >= SPLIT, vv - SPLIT, vv)

    lane = lax.iota(jnp.int32, L)
    perms = [lane ^ dd for dd in (8, 4, 2, 1)]

    def chunk_body(c, _):
      cb = c * CH
      dmas = []
      for j in range(CH // IDX_W):
        dmas.append(pltpu.async_copy(
            u_hbm.at[urix.at[pl.ds(cb + j * IDX_W, IDX_W)]],
            urows.at[pl.ds(j * IDX_W, IDX_W)], sem))
        dmas.append(pltpu.async_copy(
            v_hbm.at[vrix.at[pl.ds(cb + j * IDX_W, IDX_W)]],
            vrows.at[pl.ds(j * IDX_W, IDX_W)], sem))
      for dma in dmas:
        dma.wait()

      @plsc.parallel_loop(0, G_CH)
      def _(g):
        gb = g * L
        uiv = uix[pl.ds(cb + gb, L)]
        viv = vix[pl.ds(cb + gb, L)]
        svec = jnp.zeros((L,), jnp.float32)
        for l in range(L):
          offu = jnp.where(uiv[l] >= SPLIT, D, 0)
          offv = jnp.where(viv[l] >= SPLIT, D, 0)
          q = gb + l
          acc = jnp.zeros((L,), jnp.float32)
          for h in range(D // L):
            xu = urows[q, pl.ds(offu + h * L, L)]
            xv = vrows[q, pl.ds(offv + h * L, L)]
            acc = acc + xu * xv
          for perm in perms:
            acc = acc + acc.at[perm].get(mode="promise_in_bounds",
                                         unique_indices=True)
          svec = jnp.where(lane == l, acc, svec)
        sc[pl.ds(cb + gb, L)] = svec

    lax.fori_loop(0, N_CH, chunk_body, None)
    pltpu.sync_copy(sc, out_hbm.at[pl.ds(base, ROWS_W)])

  return k(u_idx, v_idx, Upk, Vpk)


def _tc_loss(scores):
  x = scores.reshape(B_TOT // 128, 128)
  pos_rows = B_POS // 128

  def body(x_ref, o_ref):
    xv = x_ref[...]
    row = lax.broadcasted_iota(jnp.int32, xv.shape, 0)
    sgn = jnp.where(row < pos_rows, 1.0, -1.0)
    o_ref[0, 0] = -jnp.sum(jax.nn.log_sigmoid(xv * sgn))

  out = pl.pallas_call(
      body,
      out_shape=jax.ShapeDtypeStruct((1, 1), jnp.float32),
      out_specs=pl.BlockSpec(memory_space=pltpu.SMEM),
  )(x)
  return out[0, 0]


@jax.jit
def kernel(pos_u, pos_v, neg_u, neg_v, U, V):
  u_idx = jnp.concatenate([pos_u, neg_u]).astype(jnp.int32)
  v_idx = jnp.concatenate([pos_v, neg_v]).astype(jnp.int32)
  Ut, Vt = U.T, V.T
  tail = lambda T: jnp.pad(T[:, TAILS:], ((0, 0), (0, TAILS + 2 * TBLK - R_TAB)))
  Upk, Vpk = _tc_repack(Ut, Vt, tail(Ut), tail(Vt))
  scores = _sc_scores(u_idx, v_idx, Upk, Vpk)
  return _tc_loss(scores)


# speedup vs baseline: 1.7243x; 1.7243x over previous
import functools

import jax
import jax.numpy as jnp
from jax import lax
from jax.experimental import pallas as pl
from jax.experimental.pallas import tpu as pltpu
from jax.experimental.pallas import tpu_sc as plsc

B_POS = 16384
B_NEG = 81920
B_TOT = B_POS + B_NEG
R_TAB = 1999999
D = 64
L = 16
IDX_W = 128
PK = 128


NC = 2
NS = 16
NW = NC * NS

NBLK = 15625
BLK_T = (NBLK + NW - 1) // NW
LAST_START = R_TAB - IDX_W

ROWS_W = B_TOT // NW
CH = 256
N_CH = ROWS_W // CH
G_CH = CH // L


TBLK = 2048
GRID_B = 489
SPLIT = TBLK * GRID_B
NSRC_B = R_TAB // TBLK
SAFE_B = (R_TAB - SPLIT) // TBLK
TAILS = SPLIT + SAFE_B * TBLK - TBLK


def _tc_repack(Ut, Vt, Utail, Vtail):
  def body(ut_ref, ub_ref, utl_ref, vt_ref, vb_ref, vtl_ref, uo_ref, vo_ref):
    b = pl.program_id(0)
    use_tail = b >= SAFE_B
    ub = jnp.where(use_tail, utl_ref[...], ub_ref[...])
    vb = jnp.where(use_tail, vtl_ref[...], vb_ref[...])
    uo_ref[...] = jnp.concatenate([ut_ref[...].T, ub.T], axis=1)
    vo_ref[...] = jnp.concatenate([vt_ref[...].T, vb.T], axis=1)

  top_spec = pl.BlockSpec((D, TBLK), lambda b: (0, b))
  bot_spec = pl.BlockSpec(
      (D, TBLK), lambda b: (0, jnp.minimum(GRID_B + b, NSRC_B - 1)))
  tail_spec = pl.BlockSpec(
      (D, TBLK), lambda b: (0, jnp.clip(b - (SAFE_B - 1), 0, 1)))
  out_spec = pl.BlockSpec((TBLK, PK), lambda b: (b, 0))
  return pl.pallas_call(
      body,
      grid=(GRID_B,),
      in_specs=[top_spec, bot_spec, tail_spec,
                top_spec, bot_spec, tail_spec],
      out_specs=[out_spec, out_spec],
      out_shape=[jax.ShapeDtypeStruct((SPLIT, PK), jnp.float32)] * 2,
  )(Ut, Ut, Utail, Vt, Vt, Vtail)


def _sc_scores(u_idx, v_idx, Upk, Vpk):
  mesh = plsc.VectorSubcoreMesh(core_axis_name="c", subcore_axis_name="s")

  @functools.partial(
      pl.kernel,
      out_type=jax.ShapeDtypeStruct((B_TOT,), jnp.float32),
      mesh=mesh,
      scratch_types=[
          pltpu.VMEM((ROWS_W,), jnp.int32),
          pltpu.VMEM((ROWS_W,), jnp.int32),
          pltpu.VMEM((ROWS_W,), jnp.int32),
          pltpu.VMEM((ROWS_W,), jnp.int32),
          pltpu.VMEM((CH, PK), jnp.float32),
          pltpu.VMEM((CH, PK), jnp.float32),
          pltpu.VMEM((ROWS_W,), jnp.float32),
          pltpu.SemaphoreType.DMA,
      ],
  )
  def k(u_idx_hbm, v_idx_hbm, u_hbm, v_hbm, out_hbm,
        uix, vix, urix, vrix, urows, vrows, sc, sem):
    wid = lax.axis_index("s") * NC + lax.axis_index("c")
    base = wid * ROWS_W
    pltpu.sync_copy(u_idx_hbm.at[pl.ds(base, ROWS_W)], uix)
    pltpu.sync_copy(v_idx_hbm.at[pl.ds(base, ROWS_W)], vix)

    @plsc.parallel_loop(0, ROWS_W // L, unroll=4)
    def _(t):
      s = t * L
      uu = uix[pl.ds(s, L)]
      vv = vix[pl.ds(s, L)]
      urix[pl.ds(s, L)] = jnp.where(uu >= SPLIT, uu - SPLIT, uu)
      vrix[pl.ds(s, L)] = jnp.where(vv >= SPLIT, vv - SPLIT, vv)

    lane = lax.iota(jnp.int32, L)
    perms = [lane ^ dd for dd in (8, 4, 2, 1)]

    def chunk_body(c, _):
      cb = c * CH
      dmas = []
      for j in range(CH // IDX_W):
        dmas.append(pltpu.async_copy(
            u_hbm.at[urix.at[pl.ds(cb + j * IDX_W, IDX_W)]],
            urows.at[pl.ds(j * IDX_W, IDX_W)], sem))
        dmas.append(pltpu.async_copy(
            v_hbm.at[vrix.at[pl.ds(cb + j * IDX_W, IDX_W)]],
            vrows.at[pl.ds(j * IDX_W, IDX_W)], sem))
      for dma in dmas:
        dma.wait()

      @plsc.parallel_loop(0, G_CH)
      def _(g):
        gb = g * L
        uiv = uix[pl.ds(cb + gb, L)]
        viv = vix[pl.ds(cb + gb, L)]
        svec = jnp.zeros((L,), jnp.float32)
        for l in range(L):
          offu = jnp.where(uiv[l] >= SPLIT, D, 0)
          offv = jnp.where(viv[l] >= SPLIT, D, 0)
          q = gb + l
          acc = jnp.zeros((L,), jnp.float32)
          for h in range(D // L):
            xu = urows[q, pl.ds(offu + h * L, L)]
            xv = vrows[q, pl.ds(offv + h * L, L)]
            acc = acc + xu * xv
          for perm in perms:
            acc = acc + acc.at[perm].get(mode="promise_in_bounds",
                                         unique_indices=True)
          svec = jnp.where(lane == l, acc, svec)
        sc[pl.ds(cb + gb, L)] = svec

    lax.fori_loop(0, N_CH, chunk_body, None)
    pltpu.sync_copy(sc, out_hbm.at[pl.ds(base, ROWS_W)])

  return k(u_idx, v_idx, Upk, Vpk)


def _tc_loss(scores):
  x = scores.reshape(B_TOT // 128, 128)
  pos_rows = B_POS // 128

  def body(x_ref, o_ref):
    xv = x_ref[...]
    row = lax.broadcasted_iota(jnp.int32, xv.shape, 0)
    sgn = jnp.where(row < pos_rows, 1.0, -1.0)
    o_ref[0, 0] = -jnp.sum(jax.nn.log_sigmoid(xv * sgn))

  out = pl.pallas_call(
      body,
      out_shape=jax.ShapeDtypeStruct((1, 1), jnp.float32),
      out_specs=pl.BlockSpec(memory_space=pltpu.SMEM),
  )(x)
  return out[0, 0]


@jax.jit
def kernel(pos_u, pos_v, neg_u, neg_v, U, V):
  u_idx = jnp.concatenate([pos_u, neg_u]).astype(jnp.int32)
  v_idx = jnp.concatenate([pos_v, neg_v]).astype(jnp.int32)
  Ut, Vt = U.T, V.T
  tail = lambda T: jnp.pad(T[:, TAILS:], ((0, 0), (0, TAILS + 2 * TBLK - R_TAB)))
  Upk, Vpk = _tc_repack(Ut, Vt, tail(Ut), tail(Vt))
  scores = _sc_scores(u_idx, v_idx, Upk, Vpk)
  return _tc_loss(scores)


# speedup vs baseline: 2.0152x; 1.1687x over previous
import functools

import jax
import jax.numpy as jnp
from jax import lax
from jax.experimental import pallas as pl
from jax.experimental.pallas import tpu as pltpu
from jax.experimental.pallas import tpu_sc as plsc

B_POS = 16384
B_NEG = 81920
B_TOT = B_POS + B_NEG
R_TAB = 1999999
D = 64
L = 16
IDX_W = 128
PK = 128


NC = 2
NS = 16
NW = NC * NS

NBLK = 15625
BLK_T = (NBLK + NW - 1) // NW
LAST_START = R_TAB - IDX_W

ROWS_W = B_TOT // NW
CH = 256
N_CH = ROWS_W // CH
G_CH = CH // L


TBLK = 4096
GRID_B = 245
SPLIT = TBLK * GRID_B
NSRC_B = R_TAB // TBLK
SAFE_B = (R_TAB - SPLIT) // TBLK
TAILS = SPLIT + SAFE_B * TBLK - TBLK


def _tc_repack(Ut, Vt, Utail, Vtail):
  def body(ut_ref, ub_ref, utl_ref, vt_ref, vb_ref, vtl_ref, uo_ref, vo_ref):
    b = pl.program_id(0)
    use_tail = b >= SAFE_B
    ub = jnp.where(use_tail, utl_ref[...], ub_ref[...])
    vb = jnp.where(use_tail, vtl_ref[...], vb_ref[...])
    uo_ref[...] = jnp.concatenate([ut_ref[...].T, ub.T], axis=1)
    vo_ref[...] = jnp.concatenate([vt_ref[...].T, vb.T], axis=1)

  top_spec = pl.BlockSpec((D, TBLK), lambda b: (0, b))
  bot_spec = pl.BlockSpec(
      (D, TBLK), lambda b: (0, jnp.minimum(GRID_B + b, NSRC_B - 1)))
  tail_spec = pl.BlockSpec(
      (D, TBLK), lambda b: (0, jnp.clip(b - (SAFE_B - 1), 0, 1)))
  out_spec = pl.BlockSpec((TBLK, PK), lambda b: (b, 0))
  return pl.pallas_call(
      body,
      grid=(GRID_B,),
      in_specs=[top_spec, bot_spec, tail_spec,
                top_spec, bot_spec, tail_spec],
      out_specs=[out_spec, out_spec],
      out_shape=[jax.ShapeDtypeStruct((SPLIT, PK), jnp.float32)] * 2,
  )(Ut, Ut, Utail, Vt, Vt, Vtail)


def _sc_scores(u_idx, v_idx, Upk, Vpk):
  mesh = plsc.VectorSubcoreMesh(core_axis_name="c", subcore_axis_name="s")

  @functools.partial(
      pl.kernel,
      out_type=jax.ShapeDtypeStruct((B_TOT,), jnp.float32),
      mesh=mesh,
      scratch_types=[
          pltpu.VMEM((ROWS_W,), jnp.int32),
          pltpu.VMEM((ROWS_W,), jnp.int32),
          pltpu.VMEM((ROWS_W,), jnp.int32),
          pltpu.VMEM((ROWS_W,), jnp.int32),
          pltpu.VMEM((CH, PK), jnp.float32),
          pltpu.VMEM((CH, PK), jnp.float32),
          pltpu.VMEM((ROWS_W,), jnp.float32),
          pltpu.SemaphoreType.DMA,
      ],
  )
  def k(u_idx_hbm, v_idx_hbm, u_hbm, v_hbm, out_hbm,
        uix, vix, urix, vrix, urows, vrows, sc, sem):
    wid = lax.axis_index("s") * NC + lax.axis_index("c")
    base = wid * ROWS_W
    pltpu.sync_copy(u_idx_hbm.at[pl.ds(base, ROWS_W)], uix)
    pltpu.sync_copy(v_idx_hbm.at[pl.ds(base, ROWS_W)], vix)

    @plsc.parallel_loop(0, ROWS_W // L, unroll=4)
    def _(t):
      s = t * L
      uu = uix[pl.ds(s, L)]
      vv = vix[pl.ds(s, L)]
      urix[pl.ds(s, L)] = jnp.where(uu >= SPLIT, uu - SPLIT, uu)
      vrix[pl.ds(s, L)] = jnp.where(vv >= SPLIT, vv - SPLIT, vv)

    lane = lax.iota(jnp.int32, L)
    perms = [lane ^ dd for dd in (8, 4, 2, 1)]

    def chunk_body(c, _):
      cb = c * CH
      dmas = []
      for j in range(CH // IDX_W):
        dmas.append(pltpu.async_copy(
            u_hbm.at[urix.at[pl.ds(cb + j * IDX_W, IDX_W)]],
            urows.at[pl.ds(j * IDX_W, IDX_W)], sem))
        dmas.append(pltpu.async_copy(
            v_hbm.at[vrix.at[pl.ds(cb + j * IDX_W, IDX_W)]],
            vrows.at[pl.ds(j * IDX_W, IDX_W)], sem))
      for dma in dmas:
        dma.wait()

      @plsc.parallel_loop(0, G_CH)
      def _(g):
        gb = g * L
        uiv = uix[pl.ds(cb + gb, L)]
        viv = vix[pl.ds(cb + gb, L)]
        svec = jnp.zeros((L,), jnp.float32)
        for l in range(L):
          offu = jnp.where(uiv[l] >= SPLIT, D, 0)
          offv = jnp.where(viv[l] >= SPLIT, D, 0)
          q = gb + l
          acc = jnp.zeros((L,), jnp.float32)
          for h in range(D // L):
            xu = urows[q, pl.ds(offu + h * L, L)]
            xv = vrows[q, pl.ds(offv + h * L, L)]
            acc = acc + xu * xv
          for perm in perms:
            acc = acc + acc.at[perm].get(mode="promise_in_bounds",
                                         unique_indices=True)
          svec = jnp.where(lane == l, acc, svec)
        sc[pl.ds(cb + gb, L)] = svec

    lax.fori_loop(0, N_CH, chunk_body, None)
    pltpu.sync_copy(sc, out_hbm.at[pl.ds(base, ROWS_W)])

  return k(u_idx, v_idx, Upk, Vpk)


def _tc_loss(scores):
  x = scores.reshape(B_TOT // 128, 128)
  pos_rows = B_POS // 128

  def body(x_ref, o_ref):
    xv = x_ref[...]
    row = lax.broadcasted_iota(jnp.int32, xv.shape, 0)
    sgn = jnp.where(row < pos_rows, 1.0, -1.0)
    o_ref[0, 0] = -jnp.sum(jax.nn.log_sigmoid(xv * sgn))

  out = pl.pallas_call(
      body,
      out_shape=jax.ShapeDtypeStruct((1, 1), jnp.float32),
      out_specs=pl.BlockSpec(memory_space=pltpu.SMEM),
  )(x)
  return out[0, 0]


@jax.jit
def kernel(pos_u, pos_v, neg_u, neg_v, U, V):
  u_idx = jnp.concatenate([pos_u, neg_u]).astype(jnp.int32)
  v_idx = jnp.concatenate([pos_v, neg_v]).astype(jnp.int32)
  Ut, Vt = U.T, V.T
  tail = lambda T: jnp.pad(T[:, TAILS:], ((0, 0), (0, TAILS + 2 * TBLK - R_TAB)))
  Upk, Vpk = _tc_repack(Ut, Vt, tail(Ut), tail(Vt))
  scores = _sc_scores(u_idx, v_idx, Upk, Vpk)
  return _tc_loss(scores)


# speedup vs baseline: 2.0482x; 1.0164x over previous
import functools

import jax
import jax.numpy as jnp
from jax import lax
from jax.experimental import pallas as pl
from jax.experimental.pallas import tpu as pltpu
from jax.experimental.pallas import tpu_sc as plsc

B_POS = 16384
B_NEG = 81920
B_TOT = B_POS + B_NEG
R_TAB = 1999999
D = 64
L = 16
IDX_W = 128
PK = 128


NC = 2
NS = 16
NW = NC * NS

NBLK = 15625
BLK_T = (NBLK + NW - 1) // NW
LAST_START = R_TAB - IDX_W

ROWS_W = B_TOT // NW
CH = 256
N_CH = ROWS_W // CH
G_CH = CH // L


TBLK = 8192
GRID_B = 123
SPLIT = TBLK * GRID_B
NSRC_B = R_TAB // TBLK
SAFE_B = (R_TAB - SPLIT) // TBLK
TAILS = SPLIT + SAFE_B * TBLK - TBLK


def _tc_repack(Ut, Vt, Utail, Vtail):
  def body(ut_ref, ub_ref, utl_ref, vt_ref, vb_ref, vtl_ref, uo_ref, vo_ref):
    b = pl.program_id(0)
    use_tail = b >= SAFE_B
    ub = jnp.where(use_tail, utl_ref[...], ub_ref[...])
    vb = jnp.where(use_tail, vtl_ref[...], vb_ref[...])
    uo_ref[...] = jnp.concatenate([ut_ref[...].T, ub.T], axis=1)
    vo_ref[...] = jnp.concatenate([vt_ref[...].T, vb.T], axis=1)

  top_spec = pl.BlockSpec((D, TBLK), lambda b: (0, b))
  bot_spec = pl.BlockSpec(
      (D, TBLK), lambda b: (0, jnp.minimum(GRID_B + b, NSRC_B - 1)))
  tail_spec = pl.BlockSpec(
      (D, TBLK), lambda b: (0, jnp.clip(b - (SAFE_B - 1), 0, 1)))
  out_spec = pl.BlockSpec((TBLK, PK), lambda b: (b, 0))
  return pl.pallas_call(
      body,
      grid=(GRID_B,),
      in_specs=[top_spec, bot_spec, tail_spec,
                top_spec, bot_spec, tail_spec],
      out_specs=[out_spec, out_spec],
      out_shape=[jax.ShapeDtypeStruct((SPLIT, PK), jnp.float32)] * 2,
  )(Ut, Ut, Utail, Vt, Vt, Vtail)


def _sc_scores(u_idx, v_idx, Upk, Vpk):
  mesh = plsc.VectorSubcoreMesh(core_axis_name="c", subcore_axis_name="s")

  @functools.partial(
      pl.kernel,
      out_type=jax.ShapeDtypeStruct((B_TOT,), jnp.float32),
      mesh=mesh,
      scratch_types=[
          pltpu.VMEM((ROWS_W,), jnp.int32),
          pltpu.VMEM((ROWS_W,), jnp.int32),
          pltpu.VMEM((ROWS_W,), jnp.int32),
          pltpu.VMEM((ROWS_W,), jnp.int32),
          pltpu.VMEM((CH, PK), jnp.float32),
          pltpu.VMEM((CH, PK), jnp.float32),
          pltpu.VMEM((ROWS_W,), jnp.float32),
          pltpu.SemaphoreType.DMA,
      ],
  )
  def k(u_idx_hbm, v_idx_hbm, u_hbm, v_hbm, out_hbm,
        uix, vix, urix, vrix, urows, vrows, sc, sem):
    wid = lax.axis_index("s") * NC + lax.axis_index("c")
    base = wid * ROWS_W
    pltpu.sync_copy(u_idx_hbm.at[pl.ds(base, ROWS_W)], uix)
    pltpu.sync_copy(v_idx_hbm.at[pl.ds(base, ROWS_W)], vix)

    @plsc.parallel_loop(0, ROWS_W // L, unroll=4)
    def _(t):
      s = t * L
      uu = uix[pl.ds(s, L)]
      vv = vix[pl.ds(s, L)]
      urix[pl.ds(s, L)] = jnp.where(uu >= SPLIT, uu - SPLIT, uu)
      vrix[pl.ds(s, L)] = jnp.where(vv >= SPLIT, vv - SPLIT, vv)

    lane = lax.iota(jnp.int32, L)
    perms = [lane ^ dd for dd in (8, 4, 2, 1)]

    def chunk_body(c, _):
      cb = c * CH
      dmas = []
      for j in range(CH // IDX_W):
        dmas.append(pltpu.async_copy(
            u_hbm.at[urix.at[pl.ds(cb + j * IDX_W, IDX_W)]],
            urows.at[pl.ds(j * IDX_W, IDX_W)], sem))
        dmas.append(pltpu.async_copy(
            v_hbm.at[vrix.at[pl.ds(cb + j * IDX_W, IDX_W)]],
            vrows.at[pl.ds(j * IDX_W, IDX_W)], sem))
      for dma in dmas:
        dma.wait()

      @plsc.parallel_loop(0, G_CH)
      def _(g):
        gb = g * L
        uiv = uix[pl.ds(cb + gb, L)]
        viv = vix[pl.ds(cb + gb, L)]
        svec = jnp.zeros((L,), jnp.float32)
        for l in range(L):
          offu = jnp.where(uiv[l] >= SPLIT, D, 0)
          offv = jnp.where(viv[l] >= SPLIT, D, 0)
          q = gb + l
          acc = jnp.zeros((L,), jnp.float32)
          for h in range(D // L):
            xu = urows[q, pl.ds(offu + h * L, L)]
            xv = vrows[q, pl.ds(offv + h * L, L)]
            acc = acc + xu * xv
          for perm in perms:
            acc = acc + acc.at[perm].get(mode="promise_in_bounds",
                                         unique_indices=True)
          svec = jnp.where(lane == l, acc, svec)
        sc[pl.ds(cb + gb, L)] = svec

    lax.fori_loop(0, N_CH, chunk_body, None)
    pltpu.sync_copy(sc, out_hbm.at[pl.ds(base, ROWS_W)])

  return k(u_idx, v_idx, Upk, Vpk)


def _tc_loss(scores):
  x = scores.reshape(B_TOT // 128, 128)
  pos_rows = B_POS // 128

  def body(x_ref, o_ref):
    xv = x_ref[...]
    row = lax.broadcasted_iota(jnp.int32, xv.shape, 0)
    sgn = jnp.where(row < pos_rows, 1.0, -1.0)
    o_ref[0, 0] = -jnp.sum(jax.nn.log_sigmoid(xv * sgn))

  out = pl.pallas_call(
      body,
      out_shape=jax.ShapeDtypeStruct((1, 1), jnp.float32),
      out_specs=pl.BlockSpec(memory_space=pltpu.SMEM),
  )(x)
  return out[0, 0]


@jax.jit
def kernel(pos_u, pos_v, neg_u, neg_v, U, V):
  u_idx = jnp.concatenate([pos_u, neg_u]).astype(jnp.int32)
  v_idx = jnp.concatenate([pos_v, neg_v]).astype(jnp.int32)
  Ut, Vt = U.T, V.T
  tail = lambda T: jnp.pad(T[:, TAILS:], ((0, 0), (0, TAILS + 2 * TBLK - R_TAB)))
  Upk, Vpk = _tc_repack(Ut, Vt, tail(Ut), tail(Vt))
  scores = _sc_scores(u_idx, v_idx, Upk, Vpk)
  return _tc_loss(scores)
